# baseline (speedup 1.0000x reference)
import functools

import jax
import jax.numpy as jnp
from jax import lax
from jax.experimental import pallas as pl
from jax.experimental.pallas import tpu as pltpu
from jax.experimental.pallas import tpu_sc as plsc

B = 16384
HIDDEN = 512
EMBED_DIM = 128
X_DIM = 128

VP_BIG, QB_BIG, GRID_BIG, D_BIG = 25088, 3584, 7, 32
VP_PUB, QB_PUB, GRID_PUB, D_PUB = 2560, 512, 5, 16
D_YEAR = 16

NC, NS = 2, 16
NW = NC * NS
B_PER_W = B // NW
CHUNK = 128
N_CHUNKS = B_PER_W // CHUNK
N_IDX_ROWS = B // CHUNK
L = 16


def _pack_body_big(i0, i1, i2, i3, o_ref):
    z = jnp.concatenate([i0[...], i1[...], i2[...], i3[...]], axis=0)
    o_ref[...] = z.T


def _pack_big(tt):
    specs = [
        pl.BlockSpec((D_BIG, QB_BIG),
                     functools.partial(lambda k, i: (0, GRID_BIG * k + i), k))
        for k in range(4)
    ]
    return pl.pallas_call(
        _pack_body_big,
        grid=(GRID_BIG,),
        in_specs=specs,
        out_specs=pl.BlockSpec((QB_BIG, 128), lambda i: (i, 0)),
        out_shape=jax.ShapeDtypeStruct((VP_BIG, 128), jnp.float32),
    )(tt, tt, tt, tt)


def _pack_body_pub(i0, i1, i2, i3, i4, i5, i6, i7, o_ref):
    z = jnp.concatenate(
        [r[...] for r in (i0, i1, i2, i3, i4, i5, i6, i7)], axis=0)
    o_ref[...] = z.T


def _pack_pub(tt):
    specs = [
        pl.BlockSpec((D_PUB, QB_PUB),
                     functools.partial(lambda k, i: (0, GRID_PUB * k + i), k))
        for k in range(8)
    ]
    return pl.pallas_call(
        _pack_body_pub,
        grid=(GRID_PUB,),
        in_specs=specs,
        out_specs=pl.BlockSpec((QB_PUB, 128), lambda i: (i, 0)),
        out_shape=jax.ShapeDtypeStruct((VP_PUB, 128), jnp.float32),
    )(*([tt] * 8))


def _gather_body(qt_h, qa_h, qp_h, iy_h,
                 tt_h, ta_h, tp_h, ty_h,
                 xt_h, xa_h, xp_h, xy_h,
                 iv, g0, g1, g2, g3, ybuf,
                 gs0, gs1, gs2, gs3, ws0, ws1, ws2, ws3, ysem):
    gsem = (gs0, gs1, gs2, gs3)
    wsem = (ws0, ws1, ws2, ws3)
    gbuf = (g0, g1, g2, g3)
    wid = lax.axis_index("s") * NC + lax.axis_index("c")
    base = wid * B_PER_W
    row0 = wid * N_CHUNKS
    for k, idx_h in enumerate((qt_h, qa_h, qp_h, iy_h)):
        pltpu.sync_copy(idx_h.at[pl.ds(row0, N_CHUNKS)],
                        iv.at[pl.ds(k * N_CHUNKS, N_CHUNKS)])
    ycopies = [
        pltpu.async_copy(ty_h.at[iv.at[3 * N_CHUNKS + j]],
                         ybuf.at[pl.ds(j * CHUNK, CHUNK)], ysem)
        for j in range(N_CHUNKS)
    ]
    jobs = []
    for j in range(N_CHUNKS):
        jobs.append((tt_h, 0 * N_CHUNKS + j, xt_h))
    for j in range(N_CHUNKS):
        jobs.append((ta_h, 1 * N_CHUNKS + j, xa_h))
    for j in range(N_CHUNKS):
        jobs.append((tp_h, 2 * N_CHUNKS + j, xp_h))
    copies = [None] * len(jobs)
    writes = [None] * len(jobs)
    for n in range(2):
        table, ir, _ = jobs[n]
        copies[n] = pltpu.async_copy(table.at[iv.at[ir]], gbuf[n % 4],
                                     gsem[n % 4])
    for n, (table, ir, out_h) in enumerate(jobs):
        copies[n].wait()
        orow = base + (ir % N_CHUNKS) * CHUNK
        writes[n] = pltpu.async_copy(gbuf[n % 4],
                                     out_h.at[pl.ds(orow, CHUNK)],
                                     wsem[n % 4])
        nxt = n + 2
        if nxt < len(jobs):
            if nxt - 4 >= 0:
                writes[nxt - 4].wait()
            t2, ir2, _ = jobs[nxt]
            copies[nxt] = pltpu.async_copy(t2.at[iv.at[ir2]],
                                           gbuf[nxt % 4], gsem[nxt % 4])
    for n in range(len(jobs) - 4, len(jobs)):
        writes[n].wait()
    for cp in ycopies:
        cp.wait()
    pltpu.sync_copy(ybuf, xy_h.at[pl.ds(base, B_PER_W), pl.ds(0, D_YEAR)])


_gather_cache = {}


def _get_gather():
    if "k" not in _gather_cache:
        _gather_cache["k"] = pl.kernel(
            _gather_body,
            out_type=[
                jax.ShapeDtypeStruct((B, 128), jnp.float32),
                jax.ShapeDtypeStruct((B, 128), jnp.float32),
                jax.ShapeDtypeStruct((B, 128), jnp.float32),
                jax.ShapeDtypeStruct((B, 128), jnp.float32),
            ],
            mesh=plsc.VectorSubcoreMesh(core_axis_name="c",
                                        subcore_axis_name="s"),
            scratch_types=[
                pltpu.VMEM((4 * N_CHUNKS, CHUNK), jnp.int32),
                pltpu.VMEM((CHUNK, 128), jnp.float32),
                pltpu.VMEM((CHUNK, 128), jnp.float32),
                pltpu.VMEM((CHUNK, 128), jnp.float32),
                pltpu.VMEM((CHUNK, 128), jnp.float32),
                pltpu.VMEM((B_PER_W, D_YEAR), jnp.float32),
            ] + [pltpu.SemaphoreType.DMA] * 9,
            compiler_params=pltpu.CompilerParams(use_tc_tiling_on_sc=False),
        )
    return _gather_cache["k"]


BM = 2048


def _select_segments(xq_ref, si_col, width):
    nseg = 128 // width
    acc = xq_ref[:, 0:width]
    for k in range(1, nseg):
        acc = jnp.where(si_col == k, xq_ref[:, k * width:(k + 1) * width],
                        acc)
    return acc


def _mlp_body(xt_ref, xa_ref, xp_ref, xy_ref, si_ref, w1_ref, b1_ref,
              w2_ref, b2_ref, o_ref):
    x_t = _select_segments(xt_ref, si_ref[:, 0:1], 32)
    x_a = _select_segments(xa_ref, si_ref[:, 1:2], 32)
    x_p = _select_segments(xp_ref, si_ref[:, 2:3], 16)
    x_y = xy_ref[:, 0:8]
    x = jnp.concatenate([x_t, x_a, x_p, x_y], axis=1)
    h = jnp.dot(x, w1_ref[...], preferred_element_type=jnp.float32)
    h = jnp.maximum(h + b1_ref[...], 0.0)
    o_ref[...] = jnp.dot(h, w2_ref[...],
                         preferred_element_type=jnp.float32) + b2_ref[...]


def _mlp(xt, xa, xp, xy, si, w1t, b1, w2t, b2):
    full = lambda i: (i, 0)
    rep = lambda i: (0, 0)
    return pl.pallas_call(
        _mlp_body,
        grid=(B // BM,),
        in_specs=[
            pl.BlockSpec((BM, 128), full),
            pl.BlockSpec((BM, 128), full),
            pl.BlockSpec((BM, 128), full),
            pl.BlockSpec((BM, 128), full),
            pl.BlockSpec((BM, 4), full),
            pl.BlockSpec((88, HIDDEN), rep),
            pl.BlockSpec((1, HIDDEN), rep),
            pl.BlockSpec((HIDDEN, EMBED_DIM), rep),
            pl.BlockSpec((1, EMBED_DIM), rep),
        ],
        out_specs=pl.BlockSpec((BM, EMBED_DIM), full),
        out_shape=jax.ShapeDtypeStruct((B, EMBED_DIM), jnp.float32),
    )(xt, xa, xp, xy, si, w1t, b1, w2t, b2)


def kernel(book_title, book_author, book_publisher, book_year_of_publication,
           T_title, T_author, T_pub, T_year, W1, b1, W2, b2):
    it = book_title.astype(jnp.int32)
    ia = book_author.astype(jnp.int32)
    ip = book_publisher.astype(jnp.int32)
    iy = book_year_of_publication.astype(jnp.int32)
    shp = (N_IDX_ROWS, CHUNK)
    qt = (it % VP_BIG).reshape(shp)
    qa = (ia % VP_BIG).reshape(shp)
    qp = (ip % VP_PUB).reshape(shp)
    iyr = iy.reshape(shp)
    si = jnp.stack([it // VP_BIG, ia // VP_BIG, ip // VP_PUB,
                    jnp.zeros_like(it)], axis=1)
    ttq = _pack_big(T_title.T)
    taq = _pack_big(T_author.T)
    tpq = _pack_pub(T_pub.T)
    ty16 = jnp.concatenate(
        [T_year, jnp.zeros((T_year.shape[0], 8), T_year.dtype)], axis=1)
    xt, xa, xp, xy = _get_gather()(qt, qa, qp, iyr, ttq, taq, tpq, ty16)
    return _mlp(xt, xa, xp, xy, si, W1.T, b1.reshape(1, HIDDEN),
                W2.T, b2.reshape(1, EMBED_DIM))


# speedup vs baseline: 1.2394x; 1.2394x over previous
import functools

import jax
import jax.numpy as jnp
from jax import lax
from jax.experimental import pallas as pl
from jax.experimental.pallas import tpu as pltpu
from jax.experimental.pallas import tpu_sc as plsc

B = 16384
HIDDEN = 512
EMBED_DIM = 128
X_DIM = 128

VP_BIG, QB_BIG, GRID_BIG, D_BIG = 25088, 3584, 7, 32
VP_PUB, QB_PUB, GRID_PUB, D_PUB = 2560, 512, 5, 16
D_YEAR = 16

NC, NS = 2, 16
NW = NC * NS
B_PER_W = B // NW
CHUNK = 128
N_CHUNKS = B_PER_W // CHUNK
N_IDX_ROWS = B // CHUNK
L = 16


def _pack_body_big(i0, i1, i2, i3, o_ref):
    z = jnp.concatenate([i0[...], i1[...], i2[...], i3[...]], axis=0)
    o_ref[...] = z.T


def _pack_big(tt):
    specs = [
        pl.BlockSpec((D_BIG, QB_BIG),
                     functools.partial(lambda k, i: (0, GRID_BIG * k + i), k))
        for k in range(4)
    ]
    return pl.pallas_call(
        _pack_body_big,
        grid=(GRID_BIG,),
        in_specs=specs,
        out_specs=pl.BlockSpec((QB_BIG, 128), lambda i: (i, 0)),
        out_shape=jax.ShapeDtypeStruct((VP_BIG, 128), jnp.float32),
    )(tt, tt, tt, tt)


def _pack_body_pub(i0, i1, i2, i3, i4, i5, i6, i7, o_ref):
    z = jnp.concatenate(
        [r[...] for r in (i0, i1, i2, i3, i4, i5, i6, i7)], axis=0)
    o_ref[...] = z.T


def _pack_pub(tt):
    specs = [
        pl.BlockSpec((D_PUB, QB_PUB),
                     functools.partial(lambda k, i: (0, GRID_PUB * k + i), k))
        for k in range(8)
    ]
    return pl.pallas_call(
        _pack_body_pub,
        grid=(GRID_PUB,),
        in_specs=specs,
        out_specs=pl.BlockSpec((QB_PUB, 128), lambda i: (i, 0)),
        out_shape=jax.ShapeDtypeStruct((VP_PUB, 128), jnp.float32),
    )(*([tt] * 8))


def _gather_body(qt_h, qa_h, qp_h, iy_h,
                 tt_h, ta_h, tp_h, ty_h,
                 xt_h, xa_h, xp_h, xy_h,
                 iv, g0, g1, g2, g3, ybuf,
                 gs0, gs1, gs2, gs3, ws0, ws1, ws2, ws3, ysem):
    gsem = (gs0, gs1, gs2, gs3)
    wsem = (ws0, ws1, ws2, ws3)
    gbuf = (g0, g1, g2, g3)
    wid = lax.axis_index("s") * NC + lax.axis_index("c")
    base = wid * B_PER_W
    row0 = wid * N_CHUNKS
    for k, idx_h in enumerate((qt_h, qa_h, qp_h, iy_h)):
        pltpu.sync_copy(idx_h.at[pl.ds(row0, N_CHUNKS)],
                        iv.at[pl.ds(k * N_CHUNKS, N_CHUNKS)])
    ycopies = [
        pltpu.async_copy(ty_h.at[iv.at[3 * N_CHUNKS + j]],
                         ybuf.at[pl.ds(j * CHUNK, CHUNK)], ysem)
        for j in range(N_CHUNKS)
    ]
    jobs = []
    for j in range(N_CHUNKS):
        jobs.append((tt_h, 0 * N_CHUNKS + j, xt_h))
    for j in range(N_CHUNKS):
        jobs.append((ta_h, 1 * N_CHUNKS + j, xa_h))
    for j in range(N_CHUNKS):
        jobs.append((tp_h, 2 * N_CHUNKS + j, xp_h))
    copies = [None] * len(jobs)
    writes = [None] * len(jobs)
    for n in range(2):
        table, ir, _ = jobs[n]
        copies[n] = pltpu.async_copy(table.at[iv.at[ir]], gbuf[n % 4],
                                     gsem[n % 4])
    for n, (table, ir, out_h) in enumerate(jobs):
        copies[n].wait()
        orow = base + (ir % N_CHUNKS) * CHUNK
        writes[n] = pltpu.async_copy(gbuf[n % 4],
                                     out_h.at[pl.ds(orow, CHUNK)],
                                     wsem[n % 4])
        nxt = n + 2
        if nxt < len(jobs):
            if nxt - 4 >= 0:
                writes[nxt - 4].wait()
            t2, ir2, _ = jobs[nxt]
            copies[nxt] = pltpu.async_copy(t2.at[iv.at[ir2]],
                                           gbuf[nxt % 4], gsem[nxt % 4])
    for n in range(len(jobs) - 4, len(jobs)):
        writes[n].wait()
    for cp in ycopies:
        cp.wait()
    pltpu.sync_copy(ybuf, xy_h.at[pl.ds(base, B_PER_W), pl.ds(0, D_YEAR)])


_gather_cache = {}


def _get_gather():
    if "k" not in _gather_cache:
        _gather_cache["k"] = pl.kernel(
            _gather_body,
            out_type=[
                jax.ShapeDtypeStruct((B, 128), jnp.float32),
                jax.ShapeDtypeStruct((B, 128), jnp.float32),
                jax.ShapeDtypeStruct((B, 128), jnp.float32),
                jax.ShapeDtypeStruct((B, 128), jnp.float32),
            ],
            mesh=plsc.VectorSubcoreMesh(core_axis_name="c",
                                        subcore_axis_name="s"),
            scratch_types=[
                pltpu.VMEM((4 * N_CHUNKS, CHUNK), jnp.int32),
                pltpu.VMEM((CHUNK, 128), jnp.float32),
                pltpu.VMEM((CHUNK, 128), jnp.float32),
                pltpu.VMEM((CHUNK, 128), jnp.float32),
                pltpu.VMEM((CHUNK, 128), jnp.float32),
                pltpu.VMEM((B_PER_W, D_YEAR), jnp.float32),
            ] + [pltpu.SemaphoreType.DMA] * 9,
            compiler_params=pltpu.CompilerParams(use_tc_tiling_on_sc=False),
        )
    return _gather_cache["k"]


BM = 2048


def _select_segments(xq_ref, si_col, width):
    nseg = 128 // width
    acc = xq_ref[:, 0:width]
    for k in range(1, nseg):
        acc = jnp.where(si_col == k, xq_ref[:, k * width:(k + 1) * width],
                        acc)
    return acc


def _mlp_body(xt_ref, xa_ref, xp_ref, xy_ref, si_ref, w1_ref, b1_ref,
              w2_ref, b2_ref, o_ref):
    x_t = _select_segments(xt_ref, si_ref[:, 0:32], 32)
    x_a = _select_segments(xa_ref, si_ref[:, 32:64], 32)
    x_p = _select_segments(xp_ref, si_ref[:, 64:80], 16)
    x_y = xy_ref[:, 0:8]
    x = jnp.concatenate([x_t, x_a, x_p, x_y], axis=1)
    h = jnp.dot(x, w1_ref[...], preferred_element_type=jnp.float32)
    h = jnp.maximum(h + b1_ref[...], 0.0)
    o_ref[...] = jnp.dot(h, w2_ref[...],
                         preferred_element_type=jnp.float32) + b2_ref[...]


def _mlp(xt, xa, xp, xy, si, w1t, b1, w2t, b2):
    full = lambda i: (i, 0)
    rep = lambda i: (0, 0)
    return pl.pallas_call(
        _mlp_body,
        grid=(B // BM,),
        in_specs=[
            pl.BlockSpec((BM, 128), full),
            pl.BlockSpec((BM, 128), full),
            pl.BlockSpec((BM, 128), full),
            pl.BlockSpec((BM, 128), full),
            pl.BlockSpec((BM, 128), full),
            pl.BlockSpec((88, HIDDEN), rep),
            pl.BlockSpec((1, HIDDEN), rep),
            pl.BlockSpec((HIDDEN, EMBED_DIM), rep),
            pl.BlockSpec((1, EMBED_DIM), rep),
        ],
        out_specs=pl.BlockSpec((BM, EMBED_DIM), full),
        out_shape=jax.ShapeDtypeStruct((B, EMBED_DIM), jnp.float32),
    )(xt, xa, xp, xy, si, w1t, b1, w2t, b2)


def kernel(book_title, book_author, book_publisher, book_year_of_publication,
           T_title, T_author, T_pub, T_year, W1, b1, W2, b2):
    it = book_title.astype(jnp.int32)
    ia = book_author.astype(jnp.int32)
    ip = book_publisher.astype(jnp.int32)
    iy = book_year_of_publication.astype(jnp.int32)
    shp = (N_IDX_ROWS, CHUNK)
    qt = (it % VP_BIG).reshape(shp)
    qa = (ia % VP_BIG).reshape(shp)
    qp = (ip % VP_PUB).reshape(shp)
    iyr = iy.reshape(shp)
    si = jnp.repeat(
        jnp.stack([it // VP_BIG, ia // VP_BIG, ip // VP_PUB,
                   jnp.zeros_like(it)], axis=1), 32, axis=1)
    ttq = _pack_big(T_title.T)
    taq = _pack_big(T_author.T)
    tpq = _pack_pub(T_pub.T)
    ty16 = jnp.concatenate(
        [T_year, jnp.zeros((T_year.shape[0], 8), T_year.dtype)], axis=1)
    xt, xa, xp, xy = _get_gather()(qt, qa, qp, iyr, ttq, taq, tpq, ty16)
    return _mlp(xt, xa, xp, xy, si, W1.T, b1.reshape(1, HIDDEN),
                W2.T, b2.reshape(1, EMBED_DIM))
